# baseline (speedup 1.0000x reference)
import functools

import jax
import jax.numpy as jnp
from jax import lax
from jax.experimental import pallas as pl
from jax.experimental.pallas import tpu as pltpu
from jax.experimental.pallas import tpu_sc as plsc

_B = 16384
_MF = 8
_MLP_D = 32

_NC = 2
_NS = 16
_NW = _NC * _NS
_BPW = _B // _NW
_CHUNK = 128
_NCHUNK = _BPW // _CHUNK


def _gather_body(uid_hbm, iid_hbm, gu_hbm, gi_hbm, mu_hbm, mi_hbm,
                 out_gu, out_gi, out_mu, out_mi,
                 idx_u, idx_i, rows_gu, rows_gi, rows_mu, rows_mi, sem):
    wid = lax.axis_index("s") * _NC + lax.axis_index("c")
    base = wid * _BPW
    pltpu.sync_copy(uid_hbm.at[wid], idx_u)
    pltpu.sync_copy(iid_hbm.at[wid], idx_i)
    cps = []
    for c in range(_NCHUNK):
        sl = pl.ds(c * _CHUNK, _CHUNK)
        cps.append(pltpu.async_copy(gu_hbm.at[idx_u.at[c]], rows_gu.at[sl], sem))
        cps.append(pltpu.async_copy(gi_hbm.at[idx_i.at[c]], rows_gi.at[sl], sem))
        cps.append(pltpu.async_copy(mu_hbm.at[idx_u.at[c]], rows_mu.at[sl], sem))
        cps.append(pltpu.async_copy(mi_hbm.at[idx_i.at[c]], rows_mi.at[sl], sem))
    for cp in cps:
        cp.wait()
    out_sl = pl.ds(base, _BPW)
    pltpu.sync_copy(rows_gu, out_gu.at[out_sl])
    pltpu.sync_copy(rows_gi, out_gi.at[out_sl])
    pltpu.sync_copy(rows_mu, out_mu.at[out_sl])
    pltpu.sync_copy(rows_mi, out_mi.at[out_sl])


@functools.cache
def _build_gather():
  return functools.partial(
    pl.kernel,
    mesh=plsc.VectorSubcoreMesh(core_axis_name="c", subcore_axis_name="s",
                                num_cores=_NC, num_subcores=_NS),
    out_type=[
        jax.ShapeDtypeStruct((_B, _MF), jnp.float32),
        jax.ShapeDtypeStruct((_B, _MF), jnp.float32),
        jax.ShapeDtypeStruct((_B, _MLP_D), jnp.float32),
        jax.ShapeDtypeStruct((_B, _MLP_D), jnp.float32),
    ],
    scratch_types=[
        pltpu.VMEM((_NCHUNK, _CHUNK), jnp.int32),
        pltpu.VMEM((_NCHUNK, _CHUNK), jnp.int32),
        pltpu.VMEM((_BPW, _MF), jnp.float32),
        pltpu.VMEM((_BPW, _MF), jnp.float32),
        pltpu.VMEM((_BPW, _MLP_D), jnp.float32),
        pltpu.VMEM((_BPW, _MLP_D), jnp.float32),
        pltpu.SemaphoreType.DMA,
    ],
    compiler_params=pltpu.CompilerParams(use_tc_tiling_on_sc=False),
  )(_gather_body)


def _mlp_body(gu, gi, mu, mi, w1a, w1b, b1, w2, b2, w3, b3, wpa, wpb, bp,
              out_ref):
    hp = None
    f32 = jnp.float32
    h = jnp.dot(mu[...], w1a[...], precision=hp, preferred_element_type=f32)
    h = h + jnp.dot(mi[...], w1b[...], precision=hp, preferred_element_type=f32)
    h = jnp.maximum(h + b1[...], 0.0)
    h = jnp.maximum(
        jnp.dot(h, w2[...], precision=hp, preferred_element_type=f32) + b2[...],
        0.0)
    h = jnp.maximum(
        jnp.dot(h, w3[...], precision=hp, preferred_element_type=f32) + b3[...],
        0.0)
    g = gu[...] * gi[...]
    out = jnp.sum(g * wpa[...], axis=1) + jnp.sum(h * wpb[...], axis=1)
    out_ref[...] = out + bp[0, 0]


_mlp = pl.pallas_call(
    _mlp_body,
    out_shape=jax.ShapeDtypeStruct((_B,), jnp.float32),
)


def kernel(user_ids, item_ids, gmf_user_w, gmf_item_w, mlp_user_w, mlp_item_w,
           W1, b1, W2, b2, W3, b3, Wp, bp):
    uid3 = user_ids.astype(jnp.int32).reshape(_NW, _NCHUNK, _CHUNK)
    iid3 = item_ids.astype(jnp.int32).reshape(_NW, _NCHUNK, _CHUNK)
    gu, gi, mu, mi = _build_gather()(uid3, iid3, gmf_user_w, gmf_item_w,
                                     mlp_user_w, mlp_item_w)
    w1a = W1[:_MLP_D]
    w1b = W1[_MLP_D:]
    wpa = Wp[:_MF].reshape(1, _MF)
    wpb = Wp[_MF:].reshape(1, _MF)
    return _mlp(gu, gi, mu, mi, w1a, w1b, b1.reshape(1, -1), W2,
                b2.reshape(1, -1), W3, b3.reshape(1, -1), wpa, wpb,
                bp.reshape(1, 1))


# speedup vs baseline: 4.2094x; 4.2094x over previous
import functools

import jax
import jax.numpy as jnp
from jax import lax
from jax.experimental import pallas as pl
from jax.experimental.pallas import tpu as pltpu
from jax.experimental.pallas import tpu_sc as plsc

_B = 16384
_NU = 1000000
_MF = 8
_MD = 32

_NC = 2
_NS = 16
_NW = _NC * _NS
_BPW = _B // _NW
_CHUNK = 128
_NCHUNK = _BPW // _CHUNK

_DW = 24960
_DN = 40
_DT = _NU - _DW * _DN


def _mesh():
  return plsc.VectorSubcoreMesh(core_axis_name="c", subcore_axis_name="s",
                                num_cores=_NC, num_subcores=_NS)


def _wid():
  return lax.axis_index("s") * _NC + lax.axis_index("c")


def _detile_body(gu_t, gi_t, mu_t, mi_t, out_gu, out_gi, out_mu, out_mi,
                 buf, tail, sem):
  wid = _wid()

  def do_table(tbl, out, nf):
    per_w = nf * _DN // _NW

    def main_body(i, carry):
      it = wid * per_w + i
      k = it // _DN
      j = it - k * _DN
      pltpu.sync_copy(tbl.at[k, pl.ds(j * _DW, _DW)], buf)
      pltpu.sync_copy(buf, out.at[pl.ds(k * _NU + j * _DW, _DW)])
      return carry

    lax.fori_loop(0, per_w, main_body, 0)

    @pl.when(wid < nf)
    def _():
      k = wid
      pltpu.sync_copy(tbl.at[k, pl.ds(_DW * _DN, _DT)], tail)
      pltpu.sync_copy(tail, out.at[pl.ds(k * _NU + _DW * _DN, _DT)])

  do_table(gu_t, out_gu, _MF)
  do_table(gi_t, out_gi, _MF)
  do_table(mu_t, out_mu, _MD)
  do_table(mi_t, out_mi, _MD)


@functools.cache
def _build_detile():
  return functools.partial(
      pl.kernel,
      mesh=_mesh(),
      out_type=[
          jax.ShapeDtypeStruct((_MF * _NU,), jnp.float32),
          jax.ShapeDtypeStruct((_MF * _NU,), jnp.float32),
          jax.ShapeDtypeStruct((_MD * _NU,), jnp.float32),
          jax.ShapeDtypeStruct((_MD * _NU,), jnp.float32),
      ],
      scratch_types=[
          pltpu.VMEM((_DW,), jnp.float32),
          pltpu.VMEM((_DT,), jnp.float32),
          pltpu.SemaphoreType.DMA,
      ],
      compiler_params=pltpu.CompilerParams(use_tc_tiling_on_sc=True),
  )(_detile_body)


def _gather_body(uid_hbm, iid_hbm, fgu, fgi, fmu, fmi,
                 out_gu, out_gi, out_mu, out_mi,
                 idx_u, idx_i, kidx, rows_g, rows_m, sem):
  wid = _wid()
  pltpu.sync_copy(uid_hbm.at[wid], idx_u)
  pltpu.sync_copy(iid_hbm.at[wid], idx_i)

  def do_table(flat, ids, rows, out, nf):
    cps = []
    for k in range(nf):
      for c in range(_NCHUNK):
        for v in range(_CHUNK // 16):
          sl = pl.ds(c * _CHUNK + v * 16, 16)
          kidx[k, sl] = ids[c, pl.ds(v * 16, 16)] + k * _NU
      for c in range(_NCHUNK):
        cps.append(pltpu.async_copy(
            flat.at[kidx.at[k, pl.ds(c * _CHUNK, _CHUNK)]],
            rows.at[k, pl.ds(c * _CHUNK, _CHUNK)], sem))
    for cp in cps:
      cp.wait()
    for k in range(nf):
      pltpu.sync_copy(rows.at[k], out.at[pl.ds(k * _B + wid * _BPW, _BPW)])

  do_table(fgu, idx_u, rows_g, out_gu, _MF)
  do_table(fgi, idx_i, rows_g, out_gi, _MF)
  do_table(fmu, idx_u, rows_m, out_mu, _MD)
  do_table(fmi, idx_i, rows_m, out_mi, _MD)


@functools.cache
def _build_gather():
  return functools.partial(
      pl.kernel,
      mesh=_mesh(),
      out_type=[
          jax.ShapeDtypeStruct((_MF * _B,), jnp.float32),
          jax.ShapeDtypeStruct((_MF * _B,), jnp.float32),
          jax.ShapeDtypeStruct((_MD * _B,), jnp.float32),
          jax.ShapeDtypeStruct((_MD * _B,), jnp.float32),
      ],
      scratch_types=[
          pltpu.VMEM((_NCHUNK, _CHUNK), jnp.int32),
          pltpu.VMEM((_NCHUNK, _CHUNK), jnp.int32),
          pltpu.VMEM((_MD, _BPW), jnp.int32),
          pltpu.VMEM((_MF, _BPW), jnp.float32),
          pltpu.VMEM((_MD, _BPW), jnp.float32),
          pltpu.SemaphoreType.DMA,
      ],
      compiler_params=pltpu.CompilerParams(use_tc_tiling_on_sc=False),
  )(_gather_body)


def _mlp_body(gu, gi, mu, mi, w1ta, w1tb, w2t, w3t, b1c, b2c, b3c,
              wpa, wpb, bp, out_ref):
  f32 = jnp.float32
  mu_t = mu[...].reshape(_MD, _B)
  mi_t = mi[...].reshape(_MD, _B)
  h = jnp.dot(w1ta[...], mu_t, preferred_element_type=f32)
  h = h + jnp.dot(w1tb[...], mi_t, preferred_element_type=f32)
  h = jnp.maximum(h + b1c[...], 0.0)
  h = jnp.maximum(jnp.dot(w2t[...], h, preferred_element_type=f32) + b2c[...],
                  0.0)
  h = jnp.maximum(jnp.dot(w3t[...], h, preferred_element_type=f32) + b3c[...],
                  0.0)
  g = (gu[...] * gi[...]).reshape(_MF, _B)
  out = jnp.sum(g * wpa[...], axis=0) + jnp.sum(h * wpb[...], axis=0)
  out_ref[...] = out + bp[...]


_mlp = pl.pallas_call(
    _mlp_body,
    out_shape=jax.ShapeDtypeStruct((_B,), jnp.float32),
)


def kernel(user_ids, item_ids, gmf_user_w, gmf_item_w, mlp_user_w, mlp_item_w,
           W1, b1, W2, b2, W3, b3, Wp, bp):
  uid3 = user_ids.astype(jnp.int32).reshape(_NW, _NCHUNK, _CHUNK)
  iid3 = item_ids.astype(jnp.int32).reshape(_NW, _NCHUNK, _CHUNK)
  fgu, fgi, fmu, fmi = _build_detile()(gmf_user_w.T, gmf_item_w.T,
                                       mlp_user_w.T, mlp_item_w.T)
  gu, gi, mu, mi = _build_gather()(uid3, iid3, fgu, fgi, fmu, fmi)
  w1t = W1.T
  return _mlp(gu, gi, mu, mi, w1t[:, :_MD], w1t[:, _MD:], W2.T, W3.T,
              b1.reshape(-1, 1), b2.reshape(-1, 1), b3.reshape(-1, 1),
              Wp[:_MF], Wp[_MF:], bp)


# speedup vs baseline: 4.9916x; 1.1858x over previous
import functools

import jax
import jax.numpy as jnp
from jax import lax
from jax.experimental import pallas as pl
from jax.experimental.pallas import tpu as pltpu
from jax.experimental.pallas import tpu_sc as plsc

_B = 16384
_NU = 1000000
_MF = 8
_MD = 32

_NC = 2
_NS = 16
_NW = _NC * _NS
_BPW = _B // _NW
_CHUNK = 128
_NCHUNK = _BPW // _CHUNK

_DW = 24960
_DN = 40
_DT = _NU - _DW * _DN


def _mesh():
  return plsc.VectorSubcoreMesh(core_axis_name="c", subcore_axis_name="s",
                                num_cores=_NC, num_subcores=_NS)


def _wid():
  return lax.axis_index("s") * _NC + lax.axis_index("c")


def _detile_body(gu_t, gi_t, mu_t, mi_t, out_gu, out_gi, out_mu, out_mi,
                 buf0, buf1, tail, rs0, rs1, ws0, ws1, tsem):
  wid = _wid()
  bufs = (buf0, buf1)
  rsems = (rs0, rs1)
  wsems = (ws0, ws1)

  items = []
  for tbl, out, nf in ((gu_t, out_gu, _MF), (gi_t, out_gi, _MF),
                       (mu_t, out_mu, _MD), (mi_t, out_mi, _MD)):
    per_w = nf * _DN // _NW
    for i in range(per_w):
      it = wid * per_w + i
      k = it // _DN
      j = it - k * _DN
      items.append((tbl.at[k, pl.ds(j * _DW, _DW)],
                    out.at[pl.ds(k * _NU + j * _DW, _DW)]))

  w_cp = [None, None]
  for n, (src, dst) in enumerate(items):
    b = n % 2
    if w_cp[b] is not None:
      w_cp[b].wait()
    r = pltpu.async_copy(src, bufs[b], rsems[b])
    r.wait()
    w_cp[b] = pltpu.async_copy(bufs[b], dst, wsems[b])
  for b in (0, 1):
    if w_cp[b] is not None:
      w_cp[b].wait()

  for tbl, out, nf in ((gu_t, out_gu, _MF), (gi_t, out_gi, _MF),
                       (mu_t, out_mu, _MD), (mi_t, out_mi, _MD)):
    @pl.when(wid < nf)
    def _():
      k = wid
      pltpu.sync_copy(tbl.at[k, pl.ds(_DW * _DN, _DT)], tail)
      pltpu.sync_copy(tail, out.at[pl.ds(k * _NU + _DW * _DN, _DT)])


@functools.cache
def _build_detile():
  return functools.partial(
      pl.kernel,
      mesh=_mesh(),
      out_type=[
          jax.ShapeDtypeStruct((_MF * _NU,), jnp.float32),
          jax.ShapeDtypeStruct((_MF * _NU,), jnp.float32),
          jax.ShapeDtypeStruct((_MD * _NU,), jnp.float32),
          jax.ShapeDtypeStruct((_MD * _NU,), jnp.float32),
      ],
      scratch_types=[
          pltpu.VMEM((_DW,), jnp.float32),
          pltpu.VMEM((_DW,), jnp.float32),
          pltpu.VMEM((_DT,), jnp.float32),
          pltpu.SemaphoreType.DMA,
          pltpu.SemaphoreType.DMA,
          pltpu.SemaphoreType.DMA,
          pltpu.SemaphoreType.DMA,
          pltpu.SemaphoreType.DMA,
      ],
      compiler_params=pltpu.CompilerParams(use_tc_tiling_on_sc=True),
  )(_detile_body)


def _gather_body(uid_hbm, iid_hbm, fgu, fgi, fmu, fmi,
                 out_gu, out_gi, out_mu, out_mi,
                 idx_u, idx_i, kidx_u, kidx_i,
                 rows_gu, rows_gi, rows_mu, rows_mi, sem, osem):
  wid = _wid()
  pltpu.sync_copy(uid_hbm.at[wid], idx_u)
  pltpu.sync_copy(iid_hbm.at[wid], idx_i)

  for kidx, ids in ((kidx_u, idx_u), (kidx_i, idx_i)):
    for k in range(_MD):
      for c in range(_NCHUNK):
        for v in range(_CHUNK // 16):
          sl = pl.ds(c * _CHUNK + v * 16, 16)
          kidx[k, sl] = ids[c, pl.ds(v * 16, 16)] + k * _NU

  cps = []
  for flat, kidx, rows, nf in ((fgu, kidx_u, rows_gu, _MF),
                               (fgi, kidx_i, rows_gi, _MF),
                               (fmu, kidx_u, rows_mu, _MD),
                               (fmi, kidx_i, rows_mi, _MD)):
    for k in range(nf):
      for c in range(_NCHUNK):
        cps.append(pltpu.async_copy(
            flat.at[kidx.at[k, pl.ds(c * _CHUNK, _CHUNK)]],
            rows.at[k, pl.ds(c * _CHUNK, _CHUNK)], sem))
  for cp in cps:
    cp.wait()

  ocps = []
  for rows, out, nf in ((rows_gu, out_gu, _MF), (rows_gi, out_gi, _MF),
                        (rows_mu, out_mu, _MD), (rows_mi, out_mi, _MD)):
    for k in range(nf):
      ocps.append(pltpu.async_copy(
          rows.at[k], out.at[pl.ds(k * _B + wid * _BPW, _BPW)], osem))
  for cp in ocps:
    cp.wait()


@functools.cache
def _build_gather():
  return functools.partial(
      pl.kernel,
      mesh=_mesh(),
      out_type=[
          jax.ShapeDtypeStruct((_MF * _B,), jnp.float32),
          jax.ShapeDtypeStruct((_MF * _B,), jnp.float32),
          jax.ShapeDtypeStruct((_MD * _B,), jnp.float32),
          jax.ShapeDtypeStruct((_MD * _B,), jnp.float32),
      ],
      scratch_types=[
          pltpu.VMEM((_NCHUNK, _CHUNK), jnp.int32),
          pltpu.VMEM((_NCHUNK, _CHUNK), jnp.int32),
          pltpu.VMEM((_MD, _BPW), jnp.int32),
          pltpu.VMEM((_MD, _BPW), jnp.int32),
          pltpu.VMEM((_MF, _BPW), jnp.float32),
          pltpu.VMEM((_MF, _BPW), jnp.float32),
          pltpu.VMEM((_MD, _BPW), jnp.float32),
          pltpu.VMEM((_MD, _BPW), jnp.float32),
          pltpu.SemaphoreType.DMA,
          pltpu.SemaphoreType.DMA,
      ],
      compiler_params=pltpu.CompilerParams(use_tc_tiling_on_sc=False),
  )(_gather_body)


def _mlp_body(gu, gi, mu, mi, w1ta, w1tb, w2t, w3t, b1c, b2c, b3c,
              wpa, wpb, bp, out_ref):
  f32 = jnp.float32
  mu_t = mu[...].reshape(_MD, _B)
  mi_t = mi[...].reshape(_MD, _B)
  h = jnp.dot(w1ta[...], mu_t, preferred_element_type=f32)
  h = h + jnp.dot(w1tb[...], mi_t, preferred_element_type=f32)
  h = jnp.maximum(h + b1c[...], 0.0)
  h = jnp.maximum(jnp.dot(w2t[...], h, preferred_element_type=f32) + b2c[...],
                  0.0)
  h = jnp.maximum(jnp.dot(w3t[...], h, preferred_element_type=f32) + b3c[...],
                  0.0)
  g = (gu[...] * gi[...]).reshape(_MF, _B)
  out = jnp.sum(g * wpa[...], axis=0) + jnp.sum(h * wpb[...], axis=0)
  out_ref[...] = out + bp[...]


_mlp = pl.pallas_call(
    _mlp_body,
    out_shape=jax.ShapeDtypeStruct((_B,), jnp.float32),
)


def kernel(user_ids, item_ids, gmf_user_w, gmf_item_w, mlp_user_w, mlp_item_w,
           W1, b1, W2, b2, W3, b3, Wp, bp):
  uid3 = user_ids.astype(jnp.int32).reshape(_NW, _NCHUNK, _CHUNK)
  iid3 = item_ids.astype(jnp.int32).reshape(_NW, _NCHUNK, _CHUNK)
  fgu, fgi, fmu, fmi = _build_detile()(gmf_user_w.T, gmf_item_w.T,
                                       mlp_user_w.T, mlp_item_w.T)
  gu, gi, mu, mi = _build_gather()(uid3, iid3, fgu, fgi, fmu, fmi)
  w1t = W1.T
  return _mlp(gu, gi, mu, mi, w1t[:, :_MD], w1t[:, _MD:], W2.T, W3.T,
              b1.reshape(-1, 1), b2.reshape(-1, 1), b3.reshape(-1, 1),
              Wp[:_MF], Wp[_MF:], bp)


# speedup vs baseline: 5.2873x; 1.0592x over previous
import functools

import jax
import jax.numpy as jnp
from jax import lax
from jax.experimental import pallas as pl
from jax.experimental.pallas import tpu as pltpu
from jax.experimental.pallas import tpu_sc as plsc

_B = 16384
_NU = 1000000
_MF = 8
_MD = 32

_NC = 2
_NS = 16
_NW = _NC * _NS
_BPW = _B // _NW
_CHUNK = 128
_NCHUNK = _BPW // _CHUNK

_DW = 24960
_DN = 40
_DT = _NU - _DW * _DN


def _mesh():
  return plsc.VectorSubcoreMesh(core_axis_name="c", subcore_axis_name="s",
                                num_cores=_NC, num_subcores=_NS)


def _wid():
  return lax.axis_index("s") * _NC + lax.axis_index("c")


def _detile_body(gu_t, gi_t, mu_t, mi_t, out_gu, out_gi, out_mu, out_mi,
                 buf0, buf1, buf2, tail, rs0, rs1, rs2, ws0, ws1, ws2, tsem):
  wid = _wid()
  bufs = (buf0, buf1, buf2)
  rsems = (rs0, rs1, rs2)
  wsems = (ws0, ws1, ws2)
  nb = 3

  items = []
  for tbl, out, nf in ((gu_t, out_gu, _MF), (gi_t, out_gi, _MF),
                       (mu_t, out_mu, _MD), (mi_t, out_mi, _MD)):
    per_w = nf * _DN // _NW
    for i in range(per_w):
      it = wid * per_w + i
      k = it // _DN
      j = it - k * _DN
      items.append((tbl.at[k, pl.ds(j * _DW, _DW)],
                    out.at[pl.ds(k * _NU + j * _DW, _DW)]))

  w_cp = [None] * nb
  r_cp = [None] * nb
  for n, (src, dst) in enumerate(items):
    b = n % nb
    if w_cp[b] is not None:
      w_cp[b].wait()
    r_cp[b] = pltpu.async_copy(src, bufs[b], rsems[b])
    if n >= 1:
      b1 = (n - 1) % nb
      r_cp[b1].wait()
      w_cp[b1] = pltpu.async_copy(bufs[b1], items[n - 1][1], wsems[b1])
  if items:
    n = len(items) - 1
    b = n % nb
    r_cp[b].wait()
    w_cp[b] = pltpu.async_copy(bufs[b], items[n][1], wsems[b])
  for b in range(nb):
    if w_cp[b] is not None:
      w_cp[b].wait()

  for tbl, out, nf in ((gu_t, out_gu, _MF), (gi_t, out_gi, _MF),
                       (mu_t, out_mu, _MD), (mi_t, out_mi, _MD)):
    @pl.when(wid < nf)
    def _():
      k = wid
      pltpu.sync_copy(tbl.at[k, pl.ds(_DW * _DN, _DT)], tail)
      pltpu.sync_copy(tail, out.at[pl.ds(k * _NU + _DW * _DN, _DT)])


@functools.cache
def _build_detile():
  return functools.partial(
      pl.kernel,
      mesh=_mesh(),
      out_type=[
          jax.ShapeDtypeStruct((_MF * _NU,), jnp.float32),
          jax.ShapeDtypeStruct((_MF * _NU,), jnp.float32),
          jax.ShapeDtypeStruct((_MD * _NU,), jnp.float32),
          jax.ShapeDtypeStruct((_MD * _NU,), jnp.float32),
      ],
      scratch_types=[
          pltpu.VMEM((_DW,), jnp.float32),
          pltpu.VMEM((_DW,), jnp.float32),
          pltpu.VMEM((_DW,), jnp.float32),
          pltpu.VMEM((_DT,), jnp.float32),
          pltpu.SemaphoreType.DMA,
          pltpu.SemaphoreType.DMA,
          pltpu.SemaphoreType.DMA,
          pltpu.SemaphoreType.DMA,
          pltpu.SemaphoreType.DMA,
          pltpu.SemaphoreType.DMA,
          pltpu.SemaphoreType.DMA,
      ],
      compiler_params=pltpu.CompilerParams(use_tc_tiling_on_sc=True),
  )(_detile_body)


def _gather_body(uid_hbm, iid_hbm, fgu, fgi, fmu, fmi,
                 out_gu, out_gi, out_mu, out_mi,
                 idx_u, idx_i, kidx_u, kidx_i,
                 rows_gu, rows_gi, rows_mu, rows_mi, sem, osem):
  wid = _wid()
  pltpu.sync_copy(uid_hbm.at[wid], idx_u)
  pltpu.sync_copy(iid_hbm.at[wid], idx_i)

  for kidx, ids in ((kidx_u, idx_u), (kidx_i, idx_i)):
    for k in range(_MD):
      for c in range(_NCHUNK):
        for v in range(_CHUNK // 16):
          sl = pl.ds(c * _CHUNK + v * 16, 16)
          kidx[k, sl] = ids[c, pl.ds(v * 16, 16)] + k * _NU

  cps = []
  for flat, kidx, rows, nf in ((fgu, kidx_u, rows_gu, _MF),
                               (fgi, kidx_i, rows_gi, _MF),
                               (fmu, kidx_u, rows_mu, _MD),
                               (fmi, kidx_i, rows_mi, _MD)):
    for k in range(nf):
      cps.append(pltpu.async_copy(
          flat.at[kidx.at[k]], rows.at[k], sem))
  for cp in cps:
    cp.wait()

  ocps = []
  for rows, out, nf in ((rows_gu, out_gu, _MF), (rows_gi, out_gi, _MF),
                        (rows_mu, out_mu, _MD), (rows_mi, out_mi, _MD)):
    for k in range(nf):
      ocps.append(pltpu.async_copy(
          rows.at[k], out.at[pl.ds(k * _B + wid * _BPW, _BPW)], osem))
  for cp in ocps:
    cp.wait()


@functools.cache
def _build_gather():
  return functools.partial(
      pl.kernel,
      mesh=_mesh(),
      out_type=[
          jax.ShapeDtypeStruct((_MF * _B,), jnp.float32),
          jax.ShapeDtypeStruct((_MF * _B,), jnp.float32),
          jax.ShapeDtypeStruct((_MD * _B,), jnp.float32),
          jax.ShapeDtypeStruct((_MD * _B,), jnp.float32),
      ],
      scratch_types=[
          pltpu.VMEM((_NCHUNK, _CHUNK), jnp.int32),
          pltpu.VMEM((_NCHUNK, _CHUNK), jnp.int32),
          pltpu.VMEM((_MD, _BPW), jnp.int32),
          pltpu.VMEM((_MD, _BPW), jnp.int32),
          pltpu.VMEM((_MF, _BPW), jnp.float32),
          pltpu.VMEM((_MF, _BPW), jnp.float32),
          pltpu.VMEM((_MD, _BPW), jnp.float32),
          pltpu.VMEM((_MD, _BPW), jnp.float32),
          pltpu.SemaphoreType.DMA,
          pltpu.SemaphoreType.DMA,
      ],
      compiler_params=pltpu.CompilerParams(use_tc_tiling_on_sc=False),
  )(_gather_body)


def _mlp_body(gu, gi, mu, mi, w1ta, w1tb, w2t, w3t, b1c, b2c, b3c,
              wpa, wpb, bp, out_ref):
  f32 = jnp.float32
  mu_t = mu[...].reshape(_MD, _B)
  mi_t = mi[...].reshape(_MD, _B)
  h = jnp.dot(w1ta[...], mu_t, preferred_element_type=f32)
  h = h + jnp.dot(w1tb[...], mi_t, preferred_element_type=f32)
  h = jnp.maximum(h + b1c[...], 0.0)
  h = jnp.maximum(jnp.dot(w2t[...], h, preferred_element_type=f32) + b2c[...],
                  0.0)
  h = jnp.maximum(jnp.dot(w3t[...], h, preferred_element_type=f32) + b3c[...],
                  0.0)
  g = (gu[...] * gi[...]).reshape(_MF, _B)
  out = jnp.sum(g * wpa[...], axis=0) + jnp.sum(h * wpb[...], axis=0)
  out_ref[...] = out + bp[...]


_mlp = pl.pallas_call(
    _mlp_body,
    out_shape=jax.ShapeDtypeStruct((_B,), jnp.float32),
)


def kernel(user_ids, item_ids, gmf_user_w, gmf_item_w, mlp_user_w, mlp_item_w,
           W1, b1, W2, b2, W3, b3, Wp, bp):
  uid3 = user_ids.astype(jnp.int32).reshape(_NW, _NCHUNK, _CHUNK)
  iid3 = item_ids.astype(jnp.int32).reshape(_NW, _NCHUNK, _CHUNK)
  fgu, fgi, fmu, fmi = _build_detile()(gmf_user_w.T, gmf_item_w.T,
                                       mlp_user_w.T, mlp_item_w.T)
  gu, gi, mu, mi = _build_gather()(uid3, iid3, fgu, fgi, fmu, fmi)
  w1t = W1.T
  return _mlp(gu, gi, mu, mi, w1t[:, :_MD], w1t[:, _MD:], W2.T, W3.T,
              b1.reshape(-1, 1), b2.reshape(-1, 1), b3.reshape(-1, 1),
              Wp[:_MF], Wp[_MF:], bp)


# speedup vs baseline: 5.3456x; 1.0110x over previous
import functools

import jax
import jax.numpy as jnp
from jax import lax
from jax.experimental import pallas as pl
from jax.experimental.pallas import tpu as pltpu
from jax.experimental.pallas import tpu_sc as plsc

_B = 16384
_NU = 1000000
_MF = 8
_MD = 32

_NC = 2
_NS = 16
_NW = _NC * _NS
_BPW = _B // _NW
_CHUNK = 128
_NCHUNK = _BPW // _CHUNK

_DW = 24960
_DN = 40
_DT = _NU - _DW * _DN


def _mesh():
  return plsc.VectorSubcoreMesh(core_axis_name="c", subcore_axis_name="s",
                                num_cores=_NC, num_subcores=_NS)


def _wid():
  return lax.axis_index("s") * _NC + lax.axis_index("c")


def _detile_body(gu_t, gi_t, mu_t, mi_t, out_gu, out_gi, out_mu, out_mi,
                 buf0, buf1, buf2, buf3, tail, rs0, rs1, rs2, rs3,
                 ws0, ws1, ws2, ws3, tsem):
  wid = _wid()
  bufs = (buf0, buf1, buf2, buf3)
  rsems = (rs0, rs1, rs2, rs3)
  wsems = (ws0, ws1, ws2, ws3)
  nb = 4

  items = []
  for tbl, out, nf in ((gu_t, out_gu, _MF), (gi_t, out_gi, _MF),
                       (mu_t, out_mu, _MD), (mi_t, out_mi, _MD)):
    per_w = nf * _DN // _NW
    for i in range(per_w):
      it = wid * per_w + i
      k = it // _DN
      j = it - k * _DN
      items.append((tbl.at[k, pl.ds(j * _DW, _DW)],
                    out.at[pl.ds(k * _NU + j * _DW, _DW)]))

  w_cp = [None] * nb
  r_cp = [None] * nb
  for n, (src, dst) in enumerate(items):
    b = n % nb
    if w_cp[b] is not None:
      w_cp[b].wait()
    r_cp[b] = pltpu.async_copy(src, bufs[b], rsems[b])
    if n >= 1:
      b1 = (n - 1) % nb
      r_cp[b1].wait()
      w_cp[b1] = pltpu.async_copy(bufs[b1], items[n - 1][1], wsems[b1])
  if items:
    n = len(items) - 1
    b = n % nb
    r_cp[b].wait()
    w_cp[b] = pltpu.async_copy(bufs[b], items[n][1], wsems[b])
  for b in range(nb):
    if w_cp[b] is not None:
      w_cp[b].wait()

  for tbl, out, nf in ((gu_t, out_gu, _MF), (gi_t, out_gi, _MF),
                       (mu_t, out_mu, _MD), (mi_t, out_mi, _MD)):
    @pl.when(wid < nf)
    def _():
      k = wid
      pltpu.sync_copy(tbl.at[k, pl.ds(_DW * _DN, _DT)], tail)
      pltpu.sync_copy(tail, out.at[pl.ds(k * _NU + _DW * _DN, _DT)])


@functools.cache
def _build_detile():
  return functools.partial(
      pl.kernel,
      mesh=_mesh(),
      out_type=[
          jax.ShapeDtypeStruct((_MF * _NU,), jnp.float32),
          jax.ShapeDtypeStruct((_MF * _NU,), jnp.float32),
          jax.ShapeDtypeStruct((_MD * _NU,), jnp.float32),
          jax.ShapeDtypeStruct((_MD * _NU,), jnp.float32),
      ],
      scratch_types=[
          pltpu.VMEM((_DW,), jnp.float32),
          pltpu.VMEM((_DW,), jnp.float32),
          pltpu.VMEM((_DW,), jnp.float32),
          pltpu.VMEM((_DW,), jnp.float32),
          pltpu.VMEM((_DT,), jnp.float32),
          pltpu.SemaphoreType.DMA,
          pltpu.SemaphoreType.DMA,
          pltpu.SemaphoreType.DMA,
          pltpu.SemaphoreType.DMA,
          pltpu.SemaphoreType.DMA,
          pltpu.SemaphoreType.DMA,
          pltpu.SemaphoreType.DMA,
          pltpu.SemaphoreType.DMA,
          pltpu.SemaphoreType.DMA,
      ],
      compiler_params=pltpu.CompilerParams(use_tc_tiling_on_sc=True),
  )(_detile_body)


def _gather_body(uid_hbm, iid_hbm, fgu, fgi, fmu, fmi,
                 out_gu, out_gi, out_mu, out_mi,
                 idx_u, idx_i, kidx_u, kidx_i,
                 rows_gu, rows_gi, rows_mu, rows_mi, sem, osem):
  wid = _wid()
  pltpu.sync_copy(uid_hbm.at[wid], idx_u)
  pltpu.sync_copy(iid_hbm.at[wid], idx_i)

  for kidx, ids in ((kidx_u, idx_u), (kidx_i, idx_i)):
    for k in range(_MD):
      for c in range(_NCHUNK):
        for v in range(_CHUNK // 16):
          sl = pl.ds(c * _CHUNK + v * 16, 16)
          kidx[k, sl] = ids[c, pl.ds(v * 16, 16)] + k * _NU

  tables = ((fgu, kidx_u, rows_gu, out_gu, _MF),
            (fgi, kidx_i, rows_gi, out_gi, _MF),
            (fmu, kidx_u, rows_mu, out_mu, _MD),
            (fmi, kidx_i, rows_mi, out_mi, _MD))
  cps = []
  for flat, kidx, rows, out, nf in tables:
    for k in range(nf):
      cps.append(pltpu.async_copy(flat.at[kidx.at[k]], rows.at[k], sem))
  ocps = []
  i = 0
  for flat, kidx, rows, out, nf in tables:
    for k in range(nf):
      cps[i].wait()
      i += 1
      ocps.append(pltpu.async_copy(
          rows.at[k], out.at[pl.ds(k * _B + wid * _BPW, _BPW)], osem))
  for cp in ocps:
    cp.wait()


@functools.cache
def _build_gather():
  return functools.partial(
      pl.kernel,
      mesh=_mesh(),
      out_type=[
          jax.ShapeDtypeStruct((_MF * _B,), jnp.float32),
          jax.ShapeDtypeStruct((_MF * _B,), jnp.float32),
          jax.ShapeDtypeStruct((_MD * _B,), jnp.float32),
          jax.ShapeDtypeStruct((_MD * _B,), jnp.float32),
      ],
      scratch_types=[
          pltpu.VMEM((_NCHUNK, _CHUNK), jnp.int32),
          pltpu.VMEM((_NCHUNK, _CHUNK), jnp.int32),
          pltpu.VMEM((_MD, _BPW), jnp.int32),
          pltpu.VMEM((_MD, _BPW), jnp.int32),
          pltpu.VMEM((_MF, _BPW), jnp.float32),
          pltpu.VMEM((_MF, _BPW), jnp.float32),
          pltpu.VMEM((_MD, _BPW), jnp.float32),
          pltpu.VMEM((_MD, _BPW), jnp.float32),
          pltpu.SemaphoreType.DMA,
          pltpu.SemaphoreType.DMA,
      ],
      compiler_params=pltpu.CompilerParams(use_tc_tiling_on_sc=False),
  )(_gather_body)


def _mlp_body(gu, gi, mu, mi, w1ta, w1tb, w2t, w3t, b1c, b2c, b3c,
              wpa, wpb, bp, out_ref):
  f32 = jnp.float32
  mu_t = mu[...].reshape(_MD, _B)
  mi_t = mi[...].reshape(_MD, _B)
  h = jnp.dot(w1ta[...], mu_t, preferred_element_type=f32)
  h = h + jnp.dot(w1tb[...], mi_t, preferred_element_type=f32)
  h = jnp.maximum(h + b1c[...], 0.0)
  h = jnp.maximum(jnp.dot(w2t[...], h, preferred_element_type=f32) + b2c[...],
                  0.0)
  h = jnp.maximum(jnp.dot(w3t[...], h, preferred_element_type=f32) + b3c[...],
                  0.0)
  g = (gu[...] * gi[...]).reshape(_MF, _B)
  out = jnp.sum(g * wpa[...], axis=0) + jnp.sum(h * wpb[...], axis=0)
  out_ref[...] = out + bp[...]


_mlp = pl.pallas_call(
    _mlp_body,
    out_shape=jax.ShapeDtypeStruct((_B,), jnp.float32),
)


def kernel(user_ids, item_ids, gmf_user_w, gmf_item_w, mlp_user_w, mlp_item_w,
           W1, b1, W2, b2, W3, b3, Wp, bp):
  uid3 = user_ids.astype(jnp.int32).reshape(_NW, _NCHUNK, _CHUNK)
  iid3 = item_ids.astype(jnp.int32).reshape(_NW, _NCHUNK, _CHUNK)
  fgu, fgi, fmu, fmi = _build_detile()(gmf_user_w.T, gmf_item_w.T,
                                       mlp_user_w.T, mlp_item_w.T)
  gu, gi, mu, mi = _build_gather()(uid3, iid3, fgu, fgi, fmu, fmi)
  w1t = W1.T
  return _mlp(gu, gi, mu, mi, w1t[:, :_MD], w1t[:, _MD:], W2.T, W3.T,
              b1.reshape(-1, 1), b2.reshape(-1, 1), b3.reshape(-1, 1),
              Wp[:_MF], Wp[_MF:], bp)
